# baseline (speedup 1.0000x reference)
import functools

import jax
import jax.numpy as jnp
from jax import lax
from jax.experimental import pallas as pl
from jax.experimental.pallas import tpu as pltpu
from jax.experimental.pallas import tpu_sc as plsc

N = 16384
D = 1000
DP = 1024
L = 16
NC = 2
NS = 16
NW = NC * NS
BW = N // NW
CH = 16
NCHUNK = BW // CH
NGRP = D // L
ZBLK = 512


def _zw_block(w_ref, z_ref, o_ref):
    cols = lax.broadcasted_iota(jnp.int32, (ZBLK, D), 1)
    m = cols == w_ref[...]
    o_ref[...] = jnp.sum(jnp.where(m, z_ref[...], 0.0), axis=1, keepdims=True)


def _zw_tc(w, z):
    out = pl.pallas_call(
        _zw_block,
        grid=(N // ZBLK,),
        in_specs=[
            pl.BlockSpec((ZBLK, 1), lambda i: (i, 0)),
            pl.BlockSpec((ZBLK, D), lambda i: (i, 0)),
        ],
        out_specs=pl.BlockSpec((ZBLK, 1), lambda i: (i, 0)),
        out_shape=jax.ShapeDtypeStruct((N, 1), jnp.float32),
    )(w.reshape(N, 1), z)
    return out.reshape(N)


def _compute_chunk(gb, ob, zwv, bv, kk):
    zwb = [
        plsc.load_gather(zwv, [jnp.full((L,), kk * L + r, dtype=jnp.int32)])
        for r in range(CH)
    ]

    @plsc.parallel_loop(0, NGRP, unroll=2)
    def gbody(g):
        sl = pl.ds(pl.multiple_of(g * L, L), L)
        bb = bv[sl]
        for r in range(CH):
            ob[r, sl] = gb[r, sl] * zwb[r] + bb

    sl = pl.ds(D - L, L)
    bb = bv[sl]
    for r in range(CH):
        ob[r, sl] = gb[r, sl] * zwb[r] + bb


def _sc_body(w_hbm, zw_hbm, wt_hbm, b_hbm, out_hbm,
             widx, zwv, bv, gb0, gb1, ob0, ob1,
             sg0, sg1, so0, so1):
    cid = lax.axis_index("c")
    sid = lax.axis_index("s")
    wid = sid * NC + cid
    base = wid * BW

    pltpu.sync_copy(w_hbm.at[pl.ds(base, BW)], widx)
    pltpu.sync_copy(zw_hbm.at[pl.ds(base, BW)], zwv)
    pltpu.sync_copy(b_hbm, bv)

    def start_gather(kk, gb, sg):
        idx16 = widx[pl.ds(pl.multiple_of(kk * L, L), L)]
        return pltpu.async_copy(wt_hbm.at[idx16], gb, sg)

    def wait_gather(gb, sg):
        pltpu.make_async_copy(wt_hbm.at[pl.ds(0, CH)], gb, sg).wait()

    def start_out(kk, ob, so):
        return pltpu.async_copy(ob, out_hbm.at[pl.ds(base + kk * CH, CH)], so)

    def wait_out(ob, so):
        pltpu.make_async_copy(ob, out_hbm.at[pl.ds(0, CH)], so).wait()

    start_gather(0, gb0, sg0)

    def loop_body(t, carry):
        k0 = 2 * t
        k1 = k0 + 1
        wait_gather(gb0, sg0)
        start_gather(k1, gb1, sg1)

        @pl.when(t > 0)
        def _():
            wait_out(ob0, so0)

        _compute_chunk(gb0, ob0, zwv, bv, k0)
        start_out(k0, ob0, so0)

        wait_gather(gb1, sg1)

        @pl.when(t < NCHUNK // 2 - 1)
        def _():
            start_gather(k1 + 1, gb0, sg0)

        @pl.when(t > 0)
        def _():
            wait_out(ob1, so1)

        _compute_chunk(gb1, ob1, zwv, bv, k1)
        start_out(k1, ob1, so1)
        return carry

    lax.fori_loop(0, NCHUNK // 2, loop_body, 0)
    wait_out(ob0, so0)
    wait_out(ob1, so1)


@jax.jit
def kernel(w, z, W, b):
    zw = _zw_tc(w.astype(jnp.int32), z)
    wt = jnp.pad(W.T, ((0, 0), (0, DP - D)))
    mesh = plsc.VectorSubcoreMesh(
        core_axis_name="c", subcore_axis_name="s",
        num_cores=NC, num_subcores=NS)
    f = pl.kernel(
        _sc_body,
        out_type=jax.ShapeDtypeStruct((N, D), jnp.float32),
        mesh=mesh,
        compiler_params=pltpu.CompilerParams(
            needs_layout_passes=False, use_tc_tiling_on_sc=True),
        scratch_types=[
            pltpu.VMEM((BW,), jnp.int32),
            pltpu.VMEM((BW,), jnp.float32),
            pltpu.VMEM((D,), jnp.float32),
            pltpu.VMEM((CH, DP), jnp.float32),
            pltpu.VMEM((CH, DP), jnp.float32),
            pltpu.VMEM((CH, D), jnp.float32),
            pltpu.VMEM((CH, D), jnp.float32),
            pltpu.SemaphoreType.DMA,
            pltpu.SemaphoreType.DMA,
            pltpu.SemaphoreType.DMA,
            pltpu.SemaphoreType.DMA,
        ],
    )
    return f(w.astype(jnp.int32), zw, wt, b)


# speedup vs baseline: 3.5690x; 3.5690x over previous
import functools

import jax
import jax.numpy as jnp
from jax import lax
from jax.experimental import pallas as pl
from jax.experimental.pallas import tpu as pltpu
from jax.experimental.pallas import tpu_sc as plsc

N = 16384
D = 1000
L = 16
NC = 2
NS = 16
NW = NC * NS
NU = D // 8
TPW = (NU + NW - 1) // NW
CC = 2048
NCC = N // CC
ZS = N // NS


def _body(w_hbm, zt_hbm, W_hbm, b_hbm, out_hbm,
          w_all, zw_all, bv, idxb, zw_a, zws, wbuf, wbuf1, ob0, ob1,
          sem_z, sw, sw1, so0, so1):
    cid = lax.axis_index("c")
    sid = lax.axis_index("s")
    wid = sid * NC + cid
    iota = lax.iota(jnp.int32, L)

    pltpu.sync_copy(w_hbm, w_all)
    pltpu.sync_copy(b_hbm, bv)
    abase = sid * ZS
    for g in range(ZS // L):
        off = pl.multiple_of(abase + g * L, L)
        wg = w_all[pl.ds(off, L)]
        ivec = abase + g * L + iota
        idxb[g // 8, pl.ds((g % 8) * L, L)] = (
            ((wg >> 3) << 17) + ((ivec >> 7) << 10) + ((wg & 7) << 7)
            + (ivec & 127))
    zcopies = [
        pltpu.async_copy(zt_hbm.at[idxb.at[q]], zw_a.at[pl.ds(q * 128, 128)],
                         sem_z)
        for q in range(ZS // 128)
    ]
    for c in zcopies:
        c.wait()
    pltpu.sync_copy(zw_a, zws.at[pl.ds(pl.multiple_of(abase, 8), ZS)])
    plsc.subcore_barrier()
    pltpu.sync_copy(zws, zw_all)

    def wait_out(ob, so):
        pltpu.make_async_copy(
            ob, out_hbm.at[pl.ds(0, 8), pl.ds(0, CC)], so).wait()

    def start_wslab(u, wb, sw_):
        ju = pl.multiple_of(u * 8, 8)
        return pltpu.async_copy(W_hbm.at[pl.ds(ju, 8)], wb, sw_)

    def wait_wslab(wb, sw_):
        pltpu.make_async_copy(W_hbm.at[pl.ds(0, 8)], wb, sw_).wait()

    def do_unit(t, u, wb, sw_):
        ju = pl.multiple_of(u * 8, 8)
        wait_wslab(wb, sw_)
        bbj = [
            plsc.load_gather(bv, [jnp.full((L,), ju + j, dtype=jnp.int32)])
            for j in range(8)
        ]
        for cc in range(NCC):
            ob, so = (ob0, so0) if cc % 2 == 0 else (ob1, so1)
            if cc < 2:
                @pl.when(t > 0)
                def _():
                    wait_out(ob, so)
            else:
                wait_out(ob, so)

            @plsc.parallel_loop(0, CC // L, unroll=2)
            def gbody(g):
                loff = pl.multiple_of(g * L, L)
                coff = pl.multiple_of(cc * CC + g * L, L)
                wg = w_all[pl.ds(coff, L)]
                zg = zw_all[pl.ds(coff, L)]
                for j in range(8):
                    vals = plsc.load_gather(
                        wb, [jnp.full((L,), j, dtype=jnp.int32), wg])
                    ob[j, pl.ds(loff, L)] = vals * zg + bbj[j]

            pltpu.async_copy(
                ob, out_hbm.at[pl.ds(ju, 8), pl.ds(cc * CC, CC)], so)

    start_wslab(wid, wbuf, sw)

    def pair_body(p, carry):
        for s in range(2):
            t = 2 * p + s
            u = t * NW + wid
            wb, sw_ = (wbuf, sw) if s == 0 else (wbuf1, sw1)

            @pl.when(u < NU)
            def _():
                u2 = u + NW

                @pl.when(u2 < NU)
                def _():
                    start_wslab(u2, wbuf1 if s == 0 else wbuf, sw1 if s == 0 else sw)

                do_unit(t, u, wb, sw_)

        return carry

    lax.fori_loop(0, TPW // 2, pair_body, 0)
    wait_out(ob0, so0)
    wait_out(ob1, so1)


@jax.jit
def kernel(w, z, W, b):
    zt = z.T.reshape(D // 8, 8, N // 128, 128).transpose(0, 2, 1, 3).reshape(-1)
    mesh = plsc.VectorSubcoreMesh(
        core_axis_name="c", subcore_axis_name="s",
        num_cores=NC, num_subcores=NS)
    f = pl.kernel(
        _body,
        out_type=jax.ShapeDtypeStruct((D, N), jnp.float32),
        mesh=mesh,
        compiler_params=pltpu.CompilerParams(
            needs_layout_passes=False, use_tc_tiling_on_sc=True),
        scratch_types=[
            pltpu.VMEM((N,), jnp.int32),
            pltpu.VMEM((N,), jnp.float32),
            pltpu.VMEM((D,), jnp.float32),
            pltpu.VMEM((ZS // 128, 128), jnp.int32),
            pltpu.VMEM((ZS,), jnp.float32),
            pltpu.VMEM_SHARED((N,), jnp.float32),
            pltpu.VMEM((8, D), jnp.float32),
            pltpu.VMEM((8, D), jnp.float32),
            pltpu.VMEM((8, CC), jnp.float32),
            pltpu.VMEM((8, CC), jnp.float32),
            pltpu.SemaphoreType.DMA,
            pltpu.SemaphoreType.DMA,
            pltpu.SemaphoreType.DMA,
            pltpu.SemaphoreType.DMA,
            pltpu.SemaphoreType.DMA,
        ],
    )
    out_t = f(w.astype(jnp.int32), zt, W, b)
    return out_t.T
